# baseline (speedup 1.0000x reference)
import functools

import jax
import jax.numpy as jnp
from jax import lax
from jax.experimental import pallas as pl
from jax.experimental.pallas import tpu as pltpu
from jax.experimental.pallas import tpu_sc as plsc

_D_MODEL = 768
_N_EXP = 64
_TOPK = 8
_BT = 4096
_NW = 32
_LANES = 16


def _matmul_body(x_ref, w_ref, b_ref, gate_ref):
    x = x_ref[...]
    w = w_ref[...]
    gate = jax.lax.dot_general(
        w, x, (((1,), (1,)), ((), ())), preferred_element_type=jnp.float32
    )
    gate_ref[...] = gate + b_ref[...]


def _tc_gate(x, W, b):
    T = x.shape[0]
    b2 = b.reshape(_N_EXP, 1)
    return pl.pallas_call(
        _matmul_body,
        grid=(T // _BT,),
        in_specs=[
            pl.BlockSpec((_BT, _D_MODEL), lambda i: (i, 0)),
            pl.BlockSpec((_N_EXP, _D_MODEL), lambda i: (0, 0)),
            pl.BlockSpec((_N_EXP, 1), lambda i: (0, 0)),
        ],
        out_specs=pl.BlockSpec((_N_EXP, _BT), lambda i: (0, i)),
        out_shape=jax.ShapeDtypeStruct((_N_EXP, T), jnp.float32),
        compiler_params=pltpu.CompilerParams(
            dimension_semantics=("arbitrary",),
        ),
    )(x, W, b2)


def _sc_topk(gate_t):
    E, T = gate_t.shape
    tpw = T // _NW
    mesh = plsc.VectorSubcoreMesh(core_axis_name="c", subcore_axis_name="s")

    @functools.partial(
        pl.kernel,
        mesh=mesh,
        out_type=[
            jax.ShapeDtypeStruct((_TOPK, T), jnp.int32),
            jax.ShapeDtypeStruct((_TOPK, T), jnp.float32),
        ],
        scratch_types=[
            pltpu.VMEM((E, tpw), jnp.float32),
            pltpu.VMEM((_TOPK, tpw), jnp.int32),
            pltpu.VMEM((_TOPK, tpw), jnp.float32),
        ],
    )
    def sc_kernel(gate_hbm, idx_hbm, score_hbm, slab, oidx, oscore):
        wid = lax.axis_index("s") * 2 + lax.axis_index("c")
        base = wid * tpw
        pltpu.sync_copy(gate_hbm.at[:, pl.ds(base, tpw)], slab)

        neg_inf = jnp.full((_LANES,), -jnp.inf, jnp.float32)
        zeros_i = jnp.zeros((_LANES,), jnp.int32)

        def group_body(g, _):
            off = g * _LANES

            def expert_body(e, carry):
                vals, idxs = carry
                x = slab[e, pl.ds(off, _LANES)]
                ei = jnp.full((_LANES,), e, jnp.int32)
                new_vals, new_idxs = [], []
                for j in range(_TOPK):
                    c = x > vals[j]
                    nv = jnp.where(c, x, vals[j])
                    ni = jnp.where(c, ei, idxs[j])
                    x = jnp.where(c, vals[j], x)
                    ei = jnp.where(c, idxs[j], ei)
                    new_vals.append(nv)
                    new_idxs.append(ni)
                return tuple(new_vals), tuple(new_idxs)

            vals0 = (neg_inf,) * _TOPK
            idxs0 = (zeros_i,) * _TOPK
            vals, idxs = lax.fori_loop(0, E, expert_body, (vals0, idxs0))

            es = [jnp.exp(v - vals[0]) for v in vals]
            tot = es[0]
            for t in es[1:]:
                tot = tot + t
            inv = 1.0 / tot
            for j in range(_TOPK):
                oidx[j, pl.ds(off, _LANES)] = idxs[j]
                oscore[j, pl.ds(off, _LANES)] = es[j] * inv
            return 0

        lax.fori_loop(0, tpw // _LANES, group_body, 0)
        pltpu.sync_copy(oidx, idx_hbm.at[:, pl.ds(base, tpw)])
        pltpu.sync_copy(oscore, score_hbm.at[:, pl.ds(base, tpw)])

    return sc_kernel(gate_t)


def kernel(x, W, b):
    gate_t = _tc_gate(x, W, b)
    idx_t, scores_t = _sc_topk(gate_t)
    return idx_t.T.astype(jnp.int64), scores_t.T


# speedup vs baseline: 2.3448x; 2.3448x over previous
import jax
import jax.numpy as jnp
from jax.experimental import pallas as pl
from jax.experimental.pallas import tpu as pltpu

_D_MODEL = 768
_N_EXP = 64
_TOPK = 8
_BT = 4096


def _gate_body(x_ref, w_ref, b_ref, idx_ref, score_ref):
    x = x_ref[...]
    w = w_ref[...]
    gate = jax.lax.dot_general(
        w, x, (((1,), (1,)), ((), ())), preferred_element_type=jnp.float32
    )
    gate = gate + b_ref[...]

    fiota = jax.lax.broadcasted_iota(jnp.int32, gate.shape, 0).astype(jnp.float32)
    vals = gate
    top_vals, top_idx = [], []
    for k in range(_TOPK):
        m = jnp.max(vals, axis=0, keepdims=True)
        eq = vals == m
        amin = jnp.min(jnp.where(eq, fiota, 64.0), axis=0, keepdims=True)
        top_vals.append(m)
        top_idx.append(amin)
        if k < _TOPK - 1:
            vals = jnp.where(fiota == amin, -jnp.inf, vals)

    tv = jnp.concatenate(top_vals, axis=0)
    ti = jnp.concatenate(top_idx, axis=0).astype(jnp.int32)
    e = jnp.exp(tv - tv[:1])
    score_ref[...] = e / jnp.sum(e, axis=0, keepdims=True)
    idx_ref[...] = ti


def kernel(x, W, b):
    T = x.shape[0]
    b2 = b.reshape(_N_EXP, 1)
    idx_t, scores_t = pl.pallas_call(
        _gate_body,
        grid=(T // _BT,),
        in_specs=[
            pl.BlockSpec((_BT, _D_MODEL), lambda i: (i, 0)),
            pl.BlockSpec((_N_EXP, _D_MODEL), lambda i: (0, 0)),
            pl.BlockSpec((_N_EXP, 1), lambda i: (0, 0)),
        ],
        out_specs=[
            pl.BlockSpec((_TOPK, _BT), lambda i: (0, i)),
            pl.BlockSpec((_TOPK, _BT), lambda i: (0, i)),
        ],
        out_shape=[
            jax.ShapeDtypeStruct((_TOPK, T), jnp.int32),
            jax.ShapeDtypeStruct((_TOPK, T), jnp.float32),
        ],
        compiler_params=pltpu.CompilerParams(
            dimension_semantics=("parallel",),
        ),
    )(x, W, b2)
    return idx_t.T.astype(jnp.int64), scores_t.T
